# Initial kernel scaffold; baseline (speedup 1.0000x reference)
import jax
import jax.numpy as jnp
from jax.experimental import pallas as pl


def kernel(x, angles):
    raise NotImplementedError("write your pallas kernel here")


# speedup vs baseline: 16.6583x; 16.6583x over previous
import functools

import jax
import jax.numpy as jnp
import numpy as np
from jax import lax
from jax.experimental import pallas as pl
from jax.experimental.pallas import tpu as pltpu
from jax.experimental.pallas import tpu_sc as plsc

DIM = 4096
NUM_LAYERS = 12
N_TOKENS = 8192
LANES = 16
NCH = DIM // LANES
NW = 32
ROWS_PER_W = N_TOKENS // NW
RBLK = 8
NBLK = ROWS_PER_W // RBLK
NL_REG = 4


def _lane_permute(v, idx):
    dnums = lax.GatherDimensionNumbers(
        offset_dims=(), collapsed_slice_dims=(0,), start_index_map=(0,))
    return lax.gather(v, idx[:, None], dnums, slice_sizes=(1,),
                      mode=lax.GatherScatterMode.PROMISE_IN_BOUNDS)


def _expand_tables(angles):
    cos = jnp.cos(angles)
    sin = jnp.sin(angles)
    k = np.arange(DIM)
    cos_e, sin_e = [], []
    for l in range(NL_REG):
        s = 1 << l
        pidx = ((k >> (l + 1)) << l) | (k & (s - 1))
        sign = np.where(((k >> l) & 1) == 0, 1.0, -1.0).astype(np.float32)
        cos_e.append(cos[l][pidx])
        sin_e.append(sin[l][pidx] * sign)
    return (jnp.stack(cos_e), jnp.stack(sin_e),
            cos[NL_REG:], sin[NL_REG:])


@functools.partial(
    pl.kernel,
    out_type=jax.ShapeDtypeStruct((N_TOKENS, DIM), jnp.float32),
    mesh=plsc.VectorSubcoreMesh(core_axis_name="c", subcore_axis_name="s"),
    scratch_types=[
        pltpu.VMEM((RBLK, DIM), jnp.float32),
        pltpu.VMEM((NL_REG, DIM), jnp.float32),
        pltpu.VMEM((NL_REG, DIM), jnp.float32),
        pltpu.VMEM((NUM_LAYERS - NL_REG, DIM // 2), jnp.float32),
        pltpu.VMEM((NUM_LAYERS - NL_REG, DIM // 2), jnp.float32),
    ],
)
def _butterfly_sc(x_hbm, cos_e_hbm, sin_e_hbm, cos_c_hbm, sin_c_hbm, out_hbm,
                  buf, cos_e_v, sin_e_v, cos_c_v, sin_c_v):
    wid = lax.axis_index("s") * 2 + lax.axis_index("c")

    pltpu.sync_copy(cos_e_hbm, cos_e_v)
    pltpu.sync_copy(sin_e_hbm, sin_e_v)
    pltpu.sync_copy(cos_c_hbm, cos_c_v)
    pltpu.sync_copy(sin_c_hbm, sin_c_v)

    perms = [lax.iota(jnp.int32, LANES) ^ (1 << l) for l in range(NL_REG)]

    @pl.loop(0, NBLK)
    def _blocks(blk):
        row0 = wid * ROWS_PER_W + blk * RBLK
        pltpu.sync_copy(x_hbm.at[pl.ds(row0, RBLK)], buf)

        @pl.loop(0, NCH)
        def _chunks(c):
            base = c * LANES
            ce = [cos_e_v[l, pl.ds(base, LANES)] for l in range(NL_REG)]
            se = [sin_e_v[l, pl.ds(base, LANES)] for l in range(NL_REG)]
            for r in range(RBLK):
                v = buf[r, pl.ds(base, LANES)]
                for l in range(NL_REG):
                    pv = _lane_permute(v, perms[l])
                    v = ce[l] * v + se[l] * pv
                buf[r, pl.ds(base, LANES)] = v

        for l in range(NL_REG, NUM_LAYERS):
            s = 1 << l
            lcs = l - 4
            cs = 1 << lcs

            @pl.loop(0, NCH // 2)
            def _pairs(p, l=l, s=s, lcs=lcs, cs=cs):
                hi = p >> lcs
                lo = p & (cs - 1)
                i_base = hi * (2 * s) + lo * LANES
                j_base = i_base + s
                cv = cos_c_v[l - NL_REG, pl.ds(p * LANES, LANES)]
                sv = sin_c_v[l - NL_REG, pl.ds(p * LANES, LANES)]
                for r in range(RBLK):
                    xi = buf[r, pl.ds(i_base, LANES)]
                    xj = buf[r, pl.ds(j_base, LANES)]
                    buf[r, pl.ds(i_base, LANES)] = cv * xi + sv * xj
                    buf[r, pl.ds(j_base, LANES)] = cv * xj - sv * xi

        pltpu.sync_copy(buf, out_hbm.at[pl.ds(row0, RBLK)])


def kernel(x, angles):
    orig_shape = x.shape
    x2 = x.reshape(-1, DIM)
    cos_e, sin_e, cos_c, sin_c = _expand_tables(angles)
    out = _butterfly_sc(x2, cos_e, sin_e, cos_c, sin_c)
    return out.reshape(orig_shape)


# speedup vs baseline: 25.8330x; 1.5508x over previous
import functools

import jax
import jax.numpy as jnp
import numpy as np
from jax import lax
from jax.experimental import pallas as pl
from jax.experimental.pallas import tpu as pltpu
from jax.experimental.pallas import tpu_sc as plsc

DIM = 4096
NUM_LAYERS = 12
N_TOKENS = 8192
LANES = 16
NCH = DIM // LANES
NW = 32
ROWS_PER_W = N_TOKENS // NW
RBLK = 8
NBLK = ROWS_PER_W // RBLK
NL_REG = 4


def _lane_permute(v, idx):
    dnums = lax.GatherDimensionNumbers(
        offset_dims=(), collapsed_slice_dims=(0,), start_index_map=(0,))
    return lax.gather(v, idx[:, None], dnums, slice_sizes=(1,),
                      mode=lax.GatherScatterMode.PROMISE_IN_BOUNDS)


def _expand_tables(angles):
    cos = jnp.cos(angles)
    sin = jnp.sin(angles)
    k = np.arange(DIM)
    cos_e, sin_e = [], []
    for l in range(NL_REG):
        s = 1 << l
        pidx = ((k >> (l + 1)) << l) | (k & (s - 1))
        sign = np.where(((k >> l) & 1) == 0, 1.0, -1.0).astype(np.float32)
        cos_e.append(cos[l][pidx])
        sin_e.append(sin[l][pidx] * sign)
    return (jnp.stack(cos_e), jnp.stack(sin_e),
            cos[NL_REG:], sin[NL_REG:])


@functools.partial(
    pl.kernel,
    out_type=jax.ShapeDtypeStruct((N_TOKENS, DIM), jnp.float32),
    mesh=plsc.VectorSubcoreMesh(core_axis_name="c", subcore_axis_name="s"),
    scratch_types=[
        pltpu.VMEM((RBLK, DIM), jnp.float32),
        pltpu.VMEM((NL_REG, DIM), jnp.float32),
        pltpu.VMEM((NL_REG, DIM), jnp.float32),
        pltpu.VMEM((NUM_LAYERS - NL_REG, DIM // 2), jnp.float32),
        pltpu.VMEM((NUM_LAYERS - NL_REG, DIM // 2), jnp.float32),
    ],
)
def _butterfly_sc(x_hbm, cos_e_hbm, sin_e_hbm, cos_c_hbm, sin_c_hbm, out_hbm,
                  buf, cos_e_v, sin_e_v, cos_c_v, sin_c_v):
    wid = lax.axis_index("s") * 2 + lax.axis_index("c")

    pltpu.sync_copy(cos_e_hbm, cos_e_v)
    pltpu.sync_copy(sin_e_hbm, sin_e_v)
    pltpu.sync_copy(cos_c_hbm, cos_c_v)
    pltpu.sync_copy(sin_c_hbm, sin_c_v)

    perms = [lax.iota(jnp.int32, LANES) ^ (1 << l) for l in range(NL_REG)]

    @pl.loop(0, NBLK)
    def _blocks(blk):
        row0 = wid * ROWS_PER_W + blk * RBLK
        pltpu.sync_copy(x_hbm.at[pl.ds(row0, RBLK)], buf)

        @plsc.parallel_loop(0, NCH, unroll=2)
        def _chunks(c):
            base = c * LANES
            ce = [cos_e_v[l, pl.ds(base, LANES)] for l in range(NL_REG)]
            se = [sin_e_v[l, pl.ds(base, LANES)] for l in range(NL_REG)]
            for r in range(RBLK):
                v = buf[r, pl.ds(base, LANES)]
                for l in range(NL_REG):
                    pv = _lane_permute(v, perms[l])
                    v = ce[l] * v + se[l] * pv
                buf[r, pl.ds(base, LANES)] = v

        for g in range(4):
            l = NL_REG + 2 * g
            b = 2 * g
            cs = 1 << b

            @plsc.parallel_loop(0, NCH // 4, unroll=2)
            def _quads(q, l=l, b=b, cs=cs):
                hi = q >> b
                lo = q & (cs - 1)
                base0 = (hi * (4 * cs) + lo) * LANES
                toff = (hi * (2 * cs) + lo) * LANES
                step = cs * LANES
                ca0 = cos_c_v[l - NL_REG, pl.ds(toff, LANES)]
                sa0 = sin_c_v[l - NL_REG, pl.ds(toff, LANES)]
                cb0 = cos_c_v[l - NL_REG, pl.ds(toff + step, LANES)]
                sb0 = sin_c_v[l - NL_REG, pl.ds(toff + step, LANES)]
                ca1 = cos_c_v[l - NL_REG + 1, pl.ds(toff, LANES)]
                sa1 = sin_c_v[l - NL_REG + 1, pl.ds(toff, LANES)]
                cb1 = cos_c_v[l - NL_REG + 1, pl.ds(toff + step, LANES)]
                sb1 = sin_c_v[l - NL_REG + 1, pl.ds(toff + step, LANES)]
                for r in range(RBLK):
                    x0 = buf[r, pl.ds(base0, LANES)]
                    x1 = buf[r, pl.ds(base0 + step, LANES)]
                    x2 = buf[r, pl.ds(base0 + 2 * step, LANES)]
                    x3 = buf[r, pl.ds(base0 + 3 * step, LANES)]
                    y0 = ca0 * x0 + sa0 * x1
                    y1 = ca0 * x1 - sa0 * x0
                    y2 = cb0 * x2 + sb0 * x3
                    y3 = cb0 * x3 - sb0 * x2
                    buf[r, pl.ds(base0, LANES)] = ca1 * y0 + sa1 * y2
                    buf[r, pl.ds(base0 + step, LANES)] = cb1 * y1 + sb1 * y3
                    buf[r, pl.ds(base0 + 2 * step, LANES)] = ca1 * y2 - sa1 * y0
                    buf[r, pl.ds(base0 + 3 * step, LANES)] = cb1 * y3 - sb1 * y1

        pltpu.sync_copy(buf, out_hbm.at[pl.ds(row0, RBLK)])


def kernel(x, angles):
    orig_shape = x.shape
    x2 = x.reshape(-1, DIM)
    cos_e, sin_e, cos_c, sin_c = _expand_tables(angles)
    out = _butterfly_sc(x2, cos_e, sin_e, cos_c, sin_c)
    return out.reshape(orig_shape)


# speedup vs baseline: 29.5481x; 1.1438x over previous
import functools

import jax
import jax.numpy as jnp
import numpy as np
from jax import lax
from jax.experimental import pallas as pl
from jax.experimental.pallas import tpu as pltpu
from jax.experimental.pallas import tpu_sc as plsc

DIM = 4096
NUM_LAYERS = 12
N_TOKENS = 8192
LANES = 16
NCH = DIM // LANES
NW = 32
ROWS_PER_W = N_TOKENS // NW
RBLK = 4
NBLK = ROWS_PER_W // RBLK
NBUF = 3
NBLK_MAIN = 63
NL_REG = 4


def _lane_permute(v, idx):
    dnums = lax.GatherDimensionNumbers(
        offset_dims=(), collapsed_slice_dims=(0,), start_index_map=(0,))
    return lax.gather(v, idx[:, None], dnums, slice_sizes=(1,),
                      mode=lax.GatherScatterMode.PROMISE_IN_BOUNDS)


def _expand_tables(angles):
    cos = jnp.cos(angles)
    sin = jnp.sin(angles)
    k = np.arange(DIM)
    cos_e, sin_e = [], []
    for l in range(NL_REG):
        s = 1 << l
        pidx = ((k >> (l + 1)) << l) | (k & (s - 1))
        sign = np.where(((k >> l) & 1) == 0, 1.0, -1.0).astype(np.float32)
        cos_e.append(cos[l][pidx])
        sin_e.append(sin[l][pidx] * sign)
    return (jnp.stack(cos_e), jnp.stack(sin_e),
            cos[NL_REG:], sin[NL_REG:])


@functools.partial(
    pl.kernel,
    out_type=jax.ShapeDtypeStruct((N_TOKENS, DIM), jnp.float32),
    mesh=plsc.VectorSubcoreMesh(core_axis_name="c", subcore_axis_name="s"),
    scratch_types=[
        pltpu.VMEM((RBLK, DIM), jnp.float32),
        pltpu.VMEM((RBLK, DIM), jnp.float32),
        pltpu.VMEM((RBLK, DIM), jnp.float32),
        pltpu.VMEM((NL_REG, DIM), jnp.float32),
        pltpu.VMEM((NL_REG, DIM), jnp.float32),
        pltpu.VMEM((NUM_LAYERS - NL_REG, DIM // 2), jnp.float32),
        pltpu.VMEM((NUM_LAYERS - NL_REG, DIM // 2), jnp.float32),
        pltpu.SemaphoreType.DMA,
        pltpu.SemaphoreType.DMA,
        pltpu.SemaphoreType.DMA,
        pltpu.SemaphoreType.DMA,
        pltpu.SemaphoreType.DMA,
        pltpu.SemaphoreType.DMA,
    ],
)
def _butterfly_sc(x_hbm, cos_e_hbm, sin_e_hbm, cos_c_hbm, sin_c_hbm, out_hbm,
                  buf0, buf1, buf2, cos_e_v, sin_e_v, cos_c_v, sin_c_v,
                  si0, si1, si2, so0, so1, so2):
    wid = lax.axis_index("s") * 2 + lax.axis_index("c")
    bufs = (buf0, buf1, buf2)
    sem_in = (si0, si1, si2)
    sem_out = (so0, so1, so2)

    pltpu.sync_copy(cos_e_hbm, cos_e_v)
    pltpu.sync_copy(sin_e_hbm, sin_e_v)
    pltpu.sync_copy(cos_c_hbm, cos_c_v)
    pltpu.sync_copy(sin_c_hbm, sin_c_v)

    perms = [lax.iota(jnp.int32, LANES) ^ (1 << l) for l in range(NL_REG)]

    def compute_block(buf):
        @plsc.parallel_loop(0, NCH, unroll=2)
        def _chunks(c):
            base = c * LANES
            ce = [cos_e_v[l, pl.ds(base, LANES)] for l in range(NL_REG)]
            se = [sin_e_v[l, pl.ds(base, LANES)] for l in range(NL_REG)]
            for r in range(RBLK):
                v = buf[r, pl.ds(base, LANES)]
                for l in range(NL_REG):
                    pv = _lane_permute(v, perms[l])
                    v = ce[l] * v + se[l] * pv
                buf[r, pl.ds(base, LANES)] = v

        for g in range(4):
            l = NL_REG + 2 * g
            b = 2 * g
            cs = 1 << b

            @plsc.parallel_loop(0, NCH // 4, unroll=2)
            def _quads(q, l=l, b=b, cs=cs):
                hi = q >> b
                lo = q & (cs - 1)
                base0 = (hi * (4 * cs) + lo) * LANES
                toff = (hi * (2 * cs) + lo) * LANES
                step = cs * LANES
                ca0 = cos_c_v[l - NL_REG, pl.ds(toff, LANES)]
                sa0 = sin_c_v[l - NL_REG, pl.ds(toff, LANES)]
                cb0 = cos_c_v[l - NL_REG, pl.ds(toff + step, LANES)]
                sb0 = sin_c_v[l - NL_REG, pl.ds(toff + step, LANES)]
                ca1 = cos_c_v[l - NL_REG + 1, pl.ds(toff, LANES)]
                sa1 = sin_c_v[l - NL_REG + 1, pl.ds(toff, LANES)]
                cb1 = cos_c_v[l - NL_REG + 1, pl.ds(toff + step, LANES)]
                sb1 = sin_c_v[l - NL_REG + 1, pl.ds(toff + step, LANES)]
                for r in range(RBLK):
                    x0 = buf[r, pl.ds(base0, LANES)]
                    x1 = buf[r, pl.ds(base0 + step, LANES)]
                    x2 = buf[r, pl.ds(base0 + 2 * step, LANES)]
                    x3 = buf[r, pl.ds(base0 + 3 * step, LANES)]
                    y0 = ca0 * x0 + sa0 * x1
                    y1 = ca0 * x1 - sa0 * x0
                    y2 = cb0 * x2 + sb0 * x3
                    y3 = cb0 * x3 - sb0 * x2
                    buf[r, pl.ds(base0, LANES)] = ca1 * y0 + sa1 * y2
                    buf[r, pl.ds(base0 + step, LANES)] = cb1 * y1 + sb1 * y3
                    buf[r, pl.ds(base0 + 2 * step, LANES)] = ca1 * y2 - sa1 * y0
                    buf[r, pl.ds(base0 + 3 * step, LANES)] = cb1 * y3 - sb1 * y1

    def in_cp(blk, j):
        r0 = wid * ROWS_PER_W + blk * RBLK
        return pltpu.make_async_copy(
            x_hbm.at[pl.ds(r0, RBLK)], bufs[j], sem_in[j])

    def out_cp(blk, j):
        r0 = wid * ROWS_PER_W + blk * RBLK
        return pltpu.make_async_copy(
            bufs[j], out_hbm.at[pl.ds(r0, RBLK)], sem_out[j])

    for j in range(NBUF):
        in_cp(j, j).start()

    @pl.loop(0, NBLK_MAIN, step=NBUF)
    def _blocks(k):
        for j in range(NBUF):
            b = k + j
            in_cp(b, j).wait()
            compute_block(bufs[j])
            out_cp(b, j).start()
            jm = (j + NBUF - 1) % NBUF
            bm = b - 1

            @pl.when(bm >= 0)
            def _(jm=jm, bm=bm):
                out_cp(bm, jm).wait()
                bn = bm + NBUF

                @pl.when(bn < NBLK_MAIN)
                def _(jm=jm, bn=bn):
                    in_cp(bn, jm).start()

    out_cp(NBLK_MAIN - 1, (NBLK_MAIN - 1) % NBUF).wait()
    tail = NBLK - 1
    in_cp(tail, 0).start()
    in_cp(tail, 0).wait()
    compute_block(bufs[0])
    out_cp(tail, 0).start()
    out_cp(tail, 0).wait()


def kernel(x, angles):
    orig_shape = x.shape
    x2 = x.reshape(-1, DIM)
    cos_e, sin_e, cos_c, sin_c = _expand_tables(angles)
    out = _butterfly_sc(x2, cos_e, sin_e, cos_c, sin_c)
    return out.reshape(orig_shape)
